# Initial kernel scaffold; baseline (speedup 1.0000x reference)
import jax
import jax.numpy as jnp
from jax.experimental import pallas as pl


def kernel(node_features, edge_features, neighbor_indices, mask, em_W0, em_b0, em_W1, em_b1, em_W2, em_b2, ln1_w, ln1_b, d_W0, d_b0, d_W1, d_b1, ln2_w, ln2_b, eu_W0, eu_b0, eu_W1, eu_b1, eu_W2, eu_b2, ln3_w, ln3_b):
    raise NotImplementedError("write your pallas kernel here")


# speedup vs baseline: 4.5768x; 4.5768x over previous
import functools

import jax
import jax.numpy as jnp
from jax import lax
from jax.experimental import pallas as pl
from jax.experimental.pallas import tpu as pltpu
from jax.experimental.pallas import tpu_sc as plsc

N, K, D, H = 10000, 32, 128, 512
BN = 200
GRID = N // BN
NE = N * K
ECHUNK = 128
NCHUNK = NE // ECHUNK
NW = 32


def _ln(x, w, b):
    m = jnp.mean(x, axis=-1, keepdims=True)
    v = jnp.mean(jnp.square(x - m), axis=-1, keepdims=True)
    return (x - m) * lax.rsqrt(v + 1e-5) * w + b


def _gelu(x):
    return 0.5 * x * (1.0 + lax.erf(x * 0.7071067811865476))


def _prep_body(h_ref, at_ref, ct_ref, b0_ref, p_ref, q_ref):
    h = h_ref[...]
    p_ref[...] = jnp.dot(h, at_ref[...], preferred_element_type=jnp.float32) + b0_ref[...]
    q_ref[...] = jnp.dot(h, ct_ref[...], preferred_element_type=jnp.float32)


def _prep(h, At, Ct, b0):
    return pl.pallas_call(
        _prep_body,
        out_shape=(
            jax.ShapeDtypeStruct((N, D), jnp.float32),
            jax.ShapeDtypeStruct((N, D), jnp.float32),
        ),
    )(h, At, Ct, b0.reshape(1, D))


def _gather_body(table_hbm, idx_hbm, out_hbm, idx_v, rows_v, sem):
    wid = lax.axis_index("s") * 2 + lax.axis_index("c")
    nloop = (NCHUNK + NW - 1) // NW

    def body(i, carry):
        c = i * NW + wid

        @pl.when(c < NCHUNK)
        def _():
            pltpu.sync_copy(idx_hbm.at[c], idx_v)
            pltpu.async_copy(table_hbm.at[idx_v], rows_v, sem).wait()
            pltpu.sync_copy(rows_v, out_hbm.at[pl.ds(c * ECHUNK, ECHUNK)])

        return carry

    lax.fori_loop(0, nloop, body, 0)


@functools.lru_cache(maxsize=None)
def _make_gather():
    return pl.kernel(
        _gather_body,
        out_type=jax.ShapeDtypeStruct((NE, D), jnp.float32),
        mesh=plsc.VectorSubcoreMesh(core_axis_name="c", subcore_axis_name="s"),
        scratch_types=[
            pltpu.VMEM((ECHUNK,), jnp.int32),
            pltpu.VMEM((ECHUNK, D), jnp.float32),
            pltpu.SemaphoreType.DMA,
        ],
    )


def _gather(table, idx):
    return _make_gather()(table, idx)


def _phase1_body(e_ref, qj_ref, p_ref, h_ref, mask_ref,
                 bt_ref, w1t_ref, b1_ref, w2t_ref, b2_ref,
                 ln1w_ref, ln1b_ref,
                 dw0t_ref, db0_ref, dw1t_ref, db1_ref,
                 ln2w_ref, ln2b_ref, out_ref):
    e = e_ref[...].reshape(BN * K, D)
    x = jnp.dot(e, bt_ref[...], preferred_element_type=jnp.float32) + qj_ref[...]
    p = jnp.broadcast_to(p_ref[...][:, None, :], (BN, K, D)).reshape(BN * K, D)
    x = _gelu(x + p)
    x = _gelu(jnp.dot(x, w1t_ref[...], preferred_element_type=jnp.float32) + b1_ref[...])
    m = jnp.dot(x, w2t_ref[...], preferred_element_type=jnp.float32) + b2_ref[...]
    agg = jnp.sum(m.reshape(BN, K, D), axis=1) * (1.0 / 30.0)
    h1 = _ln(h_ref[...] + agg, ln1w_ref[...], ln1b_ref[...])
    d = _gelu(jnp.dot(h1, dw0t_ref[...], preferred_element_type=jnp.float32) + db0_ref[...])
    h2 = h1 + jnp.dot(d, dw1t_ref[...], preferred_element_type=jnp.float32) + db1_ref[...]
    h2 = _ln(h2, ln2w_ref[...], ln2b_ref[...])
    out_ref[...] = h2 * mask_ref[...]


def _phase1(e, qj, p, h, mask2d, Bt, W1t, b1, W2t, b2, ln1w, ln1b,
            dW0t, db0, dW1t, db1, ln2w, ln2b):
    full = lambda shape: pl.BlockSpec(shape, lambda i: (0,) * len(shape))
    return pl.pallas_call(
        _phase1_body,
        grid=(GRID,),
        in_specs=[
            pl.BlockSpec((BN, K, D), lambda i: (i, 0, 0)),
            pl.BlockSpec((BN * K, D), lambda i: (i, 0)),
            pl.BlockSpec((BN, D), lambda i: (i, 0)),
            pl.BlockSpec((BN, D), lambda i: (i, 0)),
            pl.BlockSpec((BN, 1), lambda i: (i, 0)),
            full((D, D)), full((D, D)), full((1, D)), full((D, D)), full((1, D)),
            full((1, D)), full((1, D)),
            full((D, H)), full((1, H)), full((H, D)), full((1, D)),
            full((1, D)), full((1, D)),
        ],
        out_specs=pl.BlockSpec((BN, D), lambda i: (i, 0)),
        out_shape=jax.ShapeDtypeStruct((N, D), jnp.float32),
    )(e, qj, p, h, mask2d, Bt, W1t, b1.reshape(1, D), W2t, b2.reshape(1, D),
      ln1w.reshape(1, D), ln1b.reshape(1, D),
      dW0t, db0.reshape(1, H), dW1t, db1.reshape(1, D),
      ln2w.reshape(1, D), ln2b.reshape(1, D))


def _phase2_body(e_ref, qj_ref, p_ref,
                 bt_ref, w1t_ref, b1_ref, w2t_ref, b2_ref,
                 ln3w_ref, ln3b_ref, out_ref):
    e = e_ref[...].reshape(BN * K, D)
    x = jnp.dot(e, bt_ref[...], preferred_element_type=jnp.float32) + qj_ref[...]
    p = jnp.broadcast_to(p_ref[...][:, None, :], (BN, K, D)).reshape(BN * K, D)
    x = _gelu(x + p)
    x = _gelu(jnp.dot(x, w1t_ref[...], preferred_element_type=jnp.float32) + b1_ref[...])
    m = jnp.dot(x, w2t_ref[...], preferred_element_type=jnp.float32) + b2_ref[...]
    out = _ln(e + m, ln3w_ref[...], ln3b_ref[...])
    out_ref[...] = out.reshape(BN, K, D)


def _phase2(e, qj, p, Bt, W1t, b1, W2t, b2, ln3w, ln3b):
    full = lambda shape: pl.BlockSpec(shape, lambda i: (0,) * len(shape))
    return pl.pallas_call(
        _phase2_body,
        grid=(GRID,),
        in_specs=[
            pl.BlockSpec((BN, K, D), lambda i: (i, 0, 0)),
            pl.BlockSpec((BN * K, D), lambda i: (i, 0)),
            pl.BlockSpec((BN, D), lambda i: (i, 0)),
            full((D, D)), full((D, D)), full((1, D)), full((D, D)), full((1, D)),
            full((1, D)), full((1, D)),
        ],
        out_specs=pl.BlockSpec((BN, K, D), lambda i: (i, 0, 0)),
        out_shape=jax.ShapeDtypeStruct((N, K, D), jnp.float32),
    )(e, qj, p, Bt, W1t, b1.reshape(1, D), W2t, b2.reshape(1, D),
      ln3w.reshape(1, D), ln3b.reshape(1, D))


def kernel(node_features, edge_features, neighbor_indices, mask,
           em_W0, em_b0, em_W1, em_b1, em_W2, em_b2, ln1_w, ln1_b,
           d_W0, d_b0, d_W1, d_b1, ln2_w, ln2_b,
           eu_W0, eu_b0, eu_W1, eu_b1, eu_W2, eu_b2, ln3_w, ln3_b):
    h = node_features
    e = edge_features
    idx = neighbor_indices.reshape(NCHUNK, ECHUNK)
    mask2d = mask.reshape(N, 1)

    A1t = em_W0[:, :D].T
    B1t = em_W0[:, D:2 * D].T
    C1t = em_W0[:, 2 * D:].T
    A2t = eu_W0[:, :D].T
    B2t = eu_W0[:, D:2 * D].T
    C2t = eu_W0[:, 2 * D:].T

    p1, q1 = _prep(h, A1t, C1t, em_b0)
    qj1 = _gather(q1, idx)
    h_new = _phase1(e, qj1, p1, h, mask2d,
                    B1t, em_W1.T, em_b1, em_W2.T, em_b2, ln1_w, ln1_b,
                    d_W0.T, d_b0, d_W1.T, d_b1, ln2_w, ln2_b)

    p2, q2 = _prep(h_new, A2t, C2t, eu_b0)
    qj2 = _gather(q2, idx)
    e_out = _phase2(e, qj2, p2,
                    B2t, eu_W1.T, eu_b1, eu_W2.T, eu_b2, ln3_w, ln3_b)
    return (h_new, e_out)
